# baseline (speedup 1.0000x reference)
import functools

import jax
import jax.numpy as jnp
from jax import lax
from jax.experimental import pallas as pl
from jax.experimental.pallas import tpu as pltpu
from jax.experimental.pallas import tpu_sc as plsc

B = 64
V = 100000
VP = 102400
NVREG = VP // 16
NL1 = 400
NGROUP = 25
TOPK = 50
CAP = 128
NEG = -3.4028235e38

_mesh = plsc.VectorSubcoreMesh(core_axis_name="c", subcore_axis_name="s")


def _treemax(vs):
    vs = list(vs)
    while len(vs) > 1:
        nxt = [jnp.maximum(vs[i], vs[i + 1]) for i in range(0, len(vs) - 1, 2)]
        if len(vs) % 2:
            nxt.append(vs[-1])
        vs = nxt
    return vs[0]


@functools.partial(
    pl.kernel,
    mesh=_mesh,
    compiler_params=pltpu.CompilerParams(
        needs_layout_passes=False, use_tc_tiling_on_sc=False),
    out_type=(
        jax.ShapeDtypeStruct((B * 64,), jnp.float32),
        jax.ShapeDtypeStruct((B * 16,), jnp.int32),
    ),
    scratch_types=[
        pltpu.VMEM((VP,), jnp.float32),
        pltpu.VMEM((NL1 * 16,), jnp.float32),
        pltpu.VMEM((CAP,), jnp.float32),
        pltpu.VMEM((CAP,), jnp.int32),
        pltpu.VMEM((64,), jnp.float32),
        pltpu.VMEM((64,), jnp.int32),
        pltpu.VMEM((64,), jnp.float32),
        pltpu.VMEM((64,), jnp.float32),
        pltpu.VMEM((16,), jnp.int32),
        pltpu.SMEM((1,), jnp.int32),
    ],
)
def _sc_sampler(lg_hbm, g_hbm, probs_hbm, tok_hbm,
                row_v, l1_v, cand_val, cand_idx, slot_val, slot_idx,
                g_v, probs_v, tok_v, cnt_s):
    cid = lax.axis_index("c")
    sid = lax.axis_index("s")
    wid = sid * 2 + cid
    lane = lax.broadcasted_iota(jnp.int32, (16,), 0)
    negv = jnp.full((16,), NEG, jnp.float32)

    def _row_body(rep, carry):
        row = wid + rep * 32
        pltpu.sync_copy(lg_hbm.at[pl.ds(row * V, V)], row_v.at[pl.ds(0, V)])
        pltpu.sync_copy(g_hbm.at[pl.ds(row * 64, 64)], g_v)

        for i in range((VP - V) // 16):
            row_v[pl.ds(V + i * 16, 16)] = negv

        def _l1g(g, c):
            base = g * 256
            vs = [row_v[pl.ds(base + k * 16, 16)] for k in range(16)]
            l1_v[pl.ds(g * 16, 16)] = _treemax(vs)
            return c
        lax.fori_loop(0, NL1, _l1g, 0)

        l2 = []
        for h in range(NGROUP):
            vs = [l1_v[pl.ds((h * 16 + k) * 16, 16)] for k in range(16)]
            l2.append(_treemax(vs))

        def _round(r, state):
            t0, work = state[0], list(state[1:])
            t = plsc.cummax(_treemax(work))[15]
            tv = jnp.broadcast_to(t, (16,))
            work = [jnp.where(w == tv, NEG, w) for w in work]
            return (t, *work)
        t_c = lax.fori_loop(0, TOPK, _round, (jnp.float32(NEG), *l2))[0]
        tcv = jnp.broadcast_to(t_c, (16,))

        for a in range(CAP // 16):
            cand_val[pl.ds(a * 16, 16)] = negv
            cand_idx[pl.ds(a * 16, 16)] = jnp.zeros((16,), jnp.int32)
        cnt_s[0] = 0

        def _comp(g, c):
            l1g = l1_v[pl.ds(g * 16, 16)]

            @pl.when(jnp.any(l1g >= tcv))
            def _():
                base = g * 256
                for k in range(16):
                    x = row_v[pl.ds(base + k * 16, 16)]
                    m = x >= tcv

                    @pl.when(jnp.any(m))
                    def _():
                        c0 = cnt_s[0]
                        csum = plsc.cumsum(m.astype(jnp.int32))
                        pos = jnp.broadcast_to(c0, (16,)) + csum - 1
                        safe = jnp.logical_and(m, pos < CAP)
                        gidx = jnp.broadcast_to(base + k * 16, (16,)) + lane
                        plsc.store_scatter(cand_idx, [pos], gidx, mask=safe)
                        plsc.store_scatter(cand_val, [pos], x, mask=safe)
                        cnt_s[0] = c0 + csum[15]
            return c
        lax.fori_loop(0, NL1, _comp, 0)

        cv = [cand_val[pl.ds(a * 16, 16)] for a in range(CAP // 16)]
        ci = [cand_idx[pl.ds(a * 16, 16)] for a in range(CAP // 16)]
        cnt = jnp.minimum(cnt_s[0], CAP)

        def _rank(f, ranks):
            fv = jnp.broadcast_to(f, (16,))
            vf = plsc.load_gather(cand_val, [fv])
            jf = plsc.load_gather(cand_idx, [fv])
            out = []
            for a in range(CAP // 16):
                beat = (vf > cv[a]) | ((vf == cv[a]) & (jf < ci[a]))
                out.append(ranks[a] + beat.astype(jnp.int32))
            return tuple(out)
        ranks = lax.fori_loop(
            0, cnt, _rank,
            tuple(jnp.zeros((16,), jnp.int32) for _ in range(CAP // 16)))

        for a in range(4):
            slot_val[pl.ds(a * 16, 16)] = negv
            slot_idx[pl.ds(a * 16, 16)] = jnp.zeros((16,), jnp.int32)
        for a in range(CAP // 16):
            sel = ranks[a] < 64
            plsc.store_scatter(slot_val, [ranks[a]], cv[a], mask=sel)
            plsc.store_scatter(slot_idx, [ranks[a]], ci[a], mask=sel)

        v0v = jnp.broadcast_to(slot_val[pl.ds(0, 16)][0], (16,))
        sv = [slot_val[pl.ds(a * 16, 16)] for a in range(4)]
        siv = [slot_idx[pl.ds(a * 16, 16)] for a in range(4)]
        gvv = [g_v[pl.ds(a * 16, 16)] for a in range(4)]

        psum = jnp.zeros((16,), jnp.float32)
        pvecs = []
        for a in range(4):
            glob = lane + a * 16
            p = jnp.exp(sv[a] - v0v)
            p = jnp.where(glob < TOPK, p, jnp.float32(0.0))
            pvecs.append(p)
            psum = psum + p
        Sv = jnp.broadcast_to(plsc.cumsum(psum)[15], (16,))
        for a in range(4):
            probs_v[pl.ds(a * 16, 16)] = pvecs[a] / Sv

        mvec = negv
        scs = []
        for a in range(4):
            glob = lane + a * 16
            s = (sv[a] - v0v) + gvv[a]
            s = jnp.where(glob < TOPK, s, NEG)
            scs.append(s)
            mvec = jnp.maximum(mvec, s)
        msv = jnp.broadcast_to(plsc.cummax(mvec)[15], (16,))

        selv = jnp.full((16,), 9999, jnp.int32)
        for a in range(4):
            glob = lane + a * 16
            selv = jnp.minimum(selv, jnp.where(scs[a] == msv, glob, 9999))
        sel_i = jnp.broadcast_to(-plsc.cummax(-selv)[15], (16,))

        tokv = jnp.zeros((16,), jnp.int32)
        for a in range(4):
            glob = lane + a * 16
            tokv = tokv + jnp.where(glob == sel_i, siv[a], 0)
        tok_v[pl.ds(0, 16)] = jnp.broadcast_to(plsc.cumsum(tokv)[15], (16,))

        pltpu.sync_copy(probs_v, probs_hbm.at[pl.ds(row * 64, 64)])
        pltpu.sync_copy(tok_v, tok_hbm.at[pl.ds(row * 16, 16)])
        return carry

    lax.fori_loop(0, 2, _row_body, 0)


def kernel(logits):
    lgflat = logits[:, -1].reshape(-1)
    g = jax.random.gumbel(jax.random.key(1234), (B, TOPK), jnp.float32)
    gp = jnp.zeros((B, 64), jnp.float32).at[:, :TOPK].set(g).reshape(-1)
    probs_out, tok_out = _sc_sampler(lgflat, gp)
    return (tok_out.reshape(B, 16)[:, 0],
            probs_out.reshape(B, 64)[:, :TOPK])


# speedup vs baseline: 1.2857x; 1.2857x over previous
import functools

import jax
import jax.numpy as jnp
from jax import lax
from jax.experimental import pallas as pl
from jax.experimental.pallas import tpu as pltpu
from jax.experimental.pallas import tpu_sc as plsc

B = 64
V = 100000
VP = 102400
NVREG = VP // 16
NL1 = 400
NGROUP = 25
TOPK = 50
CAP = 128
NEG = -3.4028235e38

_mesh = plsc.VectorSubcoreMesh(core_axis_name="c", subcore_axis_name="s")


def _treemax(vs):
    vs = list(vs)
    while len(vs) > 1:
        nxt = [jnp.maximum(vs[i], vs[i + 1]) for i in range(0, len(vs) - 1, 2)]
        if len(vs) % 2:
            nxt.append(vs[-1])
        vs = nxt
    return vs[0]


@functools.partial(
    pl.kernel,
    mesh=_mesh,
    compiler_params=pltpu.CompilerParams(
        needs_layout_passes=False, use_tc_tiling_on_sc=False),
    out_type=(
        jax.ShapeDtypeStruct((B * 64,), jnp.float32),
        jax.ShapeDtypeStruct((B * 16,), jnp.int32),
    ),
    scratch_types=[
        pltpu.VMEM((VP,), jnp.float32),
        pltpu.VMEM((NL1 * 16,), jnp.float32),
        pltpu.VMEM((CAP,), jnp.float32),
        pltpu.VMEM((CAP,), jnp.int32),
        pltpu.VMEM((64,), jnp.float32),
        pltpu.VMEM((64,), jnp.int32),
        pltpu.VMEM((64,), jnp.float32),
        pltpu.VMEM((64,), jnp.float32),
        pltpu.VMEM((16,), jnp.int32),
        pltpu.VMEM((64,), jnp.int32),
    ],
)
def _sc_sampler(lg_hbm, g_hbm, probs_hbm, tok_hbm,
                row_v, l1_v, cand_val, cand_idx, slot_val, slot_idx,
                g_v, probs_v, tok_v, vlist_v):
    cid = lax.axis_index("c")
    sid = lax.axis_index("s")
    wid = sid * 2 + cid
    lane = lax.broadcasted_iota(jnp.int32, (16,), 0)
    negv = jnp.full((16,), NEG, jnp.float32)

    def _row_body(rep, carry):
        row = wid + rep * 32
        pltpu.sync_copy(lg_hbm.at[row], row_v.at[pl.ds(0, V)])
        pltpu.sync_copy(g_hbm.at[pl.ds(row * 64, 64)], g_v)

        for i in range((VP - V) // 16):
            row_v[pl.ds(V + i * 16, 16)] = negv

        lane15 = lane == 15
        def _pa(g, c):
            base = g * 256
            idv = jnp.broadcast_to(g * 16, (16,))
            for k in range(16):
                x = row_v[pl.ds(base + k * 16, 16)]
                sc = plsc.cummax(x)
                plsc.store_scatter(l1_v, [idv + k], sc, mask=lane15)
            return c
        lax.fori_loop(0, NL1, _pa, 0)

        l2 = []
        for h in range(NGROUP):
            vs = [l1_v[pl.ds((h * 16 + k) * 16, 16)] for k in range(16)]
            l2.append(_treemax(vs))

        def _round(r, state):
            t0, work = state[0], list(state[1:])
            t = plsc.cummax(_treemax(work))[15]
            tv = jnp.broadcast_to(t, (16,))
            work = [jnp.where(w == tv, NEG, w) for w in work]
            return (t, *work)
        t_c = lax.fori_loop(0, TOPK, _round, (jnp.float32(NEG), *l2))[0]
        tcv = jnp.broadcast_to(t_c, (16,))

        zero16 = jnp.zeros((16,), jnp.int32)

        def _pb(j, vcnt):
            x = l1_v[pl.ds(j * 16, 16)]
            m = x >= tcv
            csum = plsc.cumsum(m.astype(jnp.int32))
            pos = vcnt + csum - 1
            safe = jnp.logical_and(m, pos < 64)
            ids = jnp.broadcast_to(j * 16, (16,)) + lane
            plsc.store_scatter(vlist_v, [pos], ids, mask=safe)
            return vcnt + plsc.all_reduce_population_count(m)
        vcnt_vec = lax.fori_loop(0, NL1, _pb, zero16)
        vcnt = jnp.minimum(vcnt_vec[0], 64)

        for a in range(CAP // 16):
            cand_val[pl.ds(a * 16, 16)] = negv
            cand_idx[pl.ds(a * 16, 16)] = jnp.zeros((16,), jnp.int32)

        def _pc(e, ccnt):
            vv = plsc.load_gather(vlist_v, [jnp.broadcast_to(e, (16,))])
            x = row_v[pl.ds(vv[0] * 16, 16)]
            m = x >= tcv
            csum = plsc.cumsum(m.astype(jnp.int32))
            pos = ccnt + csum - 1
            safe = jnp.logical_and(m, pos < CAP)
            gidx = vv * 16 + lane
            plsc.store_scatter(cand_idx, [pos], gidx, mask=safe)
            plsc.store_scatter(cand_val, [pos], x, mask=safe)
            return ccnt + plsc.all_reduce_population_count(m)
        cnt_vec = lax.fori_loop(0, vcnt, _pc, zero16)

        cv = [cand_val[pl.ds(a * 16, 16)] for a in range(CAP // 16)]
        ci = [cand_idx[pl.ds(a * 16, 16)] for a in range(CAP // 16)]
        cnt = jnp.minimum(cnt_vec[0], CAP)

        def _rank(f, ranks):
            fv = jnp.broadcast_to(f, (16,))
            vf = plsc.load_gather(cand_val, [fv])
            jf = plsc.load_gather(cand_idx, [fv])
            out = []
            for a in range(CAP // 16):
                beat = (vf > cv[a]) | ((vf == cv[a]) & (jf < ci[a]))
                out.append(ranks[a] + beat.astype(jnp.int32))
            return tuple(out)
        ranks = lax.fori_loop(
            0, cnt, _rank,
            tuple(jnp.zeros((16,), jnp.int32) for _ in range(CAP // 16)))

        for a in range(4):
            slot_val[pl.ds(a * 16, 16)] = negv
            slot_idx[pl.ds(a * 16, 16)] = jnp.zeros((16,), jnp.int32)
        for a in range(CAP // 16):
            sel = ranks[a] < 64
            plsc.store_scatter(slot_val, [ranks[a]], cv[a], mask=sel)
            plsc.store_scatter(slot_idx, [ranks[a]], ci[a], mask=sel)

        v0v = jnp.broadcast_to(slot_val[pl.ds(0, 16)][0], (16,))
        sv = [slot_val[pl.ds(a * 16, 16)] for a in range(4)]
        siv = [slot_idx[pl.ds(a * 16, 16)] for a in range(4)]
        gvv = [g_v[pl.ds(a * 16, 16)] for a in range(4)]

        psum = jnp.zeros((16,), jnp.float32)
        pvecs = []
        for a in range(4):
            glob = lane + a * 16
            p = jnp.exp(sv[a] - v0v)
            p = jnp.where(glob < TOPK, p, jnp.float32(0.0))
            pvecs.append(p)
            psum = psum + p
        Sv = jnp.broadcast_to(plsc.cumsum(psum)[15], (16,))
        for a in range(4):
            probs_v[pl.ds(a * 16, 16)] = pvecs[a] / Sv

        mvec = negv
        scs = []
        for a in range(4):
            glob = lane + a * 16
            s = (sv[a] - v0v) + gvv[a]
            s = jnp.where(glob < TOPK, s, NEG)
            scs.append(s)
            mvec = jnp.maximum(mvec, s)
        msv = jnp.broadcast_to(plsc.cummax(mvec)[15], (16,))

        selv = jnp.full((16,), 9999, jnp.int32)
        for a in range(4):
            glob = lane + a * 16
            selv = jnp.minimum(selv, jnp.where(scs[a] == msv, glob, 9999))
        sel_i = jnp.broadcast_to(-plsc.cummax(-selv)[15], (16,))

        tokv = jnp.zeros((16,), jnp.int32)
        for a in range(4):
            glob = lane + a * 16
            tokv = tokv + jnp.where(glob == sel_i, siv[a], 0)
        tok_v[pl.ds(0, 16)] = jnp.broadcast_to(plsc.cumsum(tokv)[15], (16,))

        pltpu.sync_copy(probs_v, probs_hbm.at[pl.ds(row * 64, 64)])
        pltpu.sync_copy(tok_v, tok_hbm.at[pl.ds(row * 16, 16)])
        return carry

    lax.fori_loop(0, 2, _row_body, 0)


def kernel(logits):
    lgflat = logits[:, -1]
    g = jax.random.gumbel(jax.random.key(1234), (B, TOPK), jnp.float32)
    gp = jnp.zeros((B, 64), jnp.float32).at[:, :TOPK].set(g).reshape(-1)
    probs_out, tok_out = _sc_sampler(lgflat, gp)
    return (tok_out.reshape(B, 16)[:, 0],
            probs_out.reshape(B, 64)[:, :TOPK])


# speedup vs baseline: 1.8543x; 1.4422x over previous
import functools

import jax
import jax.numpy as jnp
from jax import lax
from jax.experimental import pallas as pl
from jax.experimental.pallas import tpu as pltpu
from jax.experimental.pallas import tpu_sc as plsc

B = 64
V = 100000
VP = 102400
NVREG = VP // 16
NL1 = 400
NGROUP = 25
TOPK = 50
CAP = 128
NEG = -3.4028235e38

_mesh = plsc.VectorSubcoreMesh(core_axis_name="c", subcore_axis_name="s")


def _treemax(vs):
    vs = list(vs)
    while len(vs) > 1:
        nxt = [jnp.maximum(vs[i], vs[i + 1]) for i in range(0, len(vs) - 1, 2)]
        if len(vs) % 2:
            nxt.append(vs[-1])
        vs = nxt
    return vs[0]


@functools.partial(
    pl.kernel,
    mesh=_mesh,
    compiler_params=pltpu.CompilerParams(
        needs_layout_passes=False, use_tc_tiling_on_sc=False),
    out_type=(
        jax.ShapeDtypeStruct((B * 64,), jnp.float32),
        jax.ShapeDtypeStruct((B * 16,), jnp.int32),
    ),
    scratch_types=[
        pltpu.VMEM((VP,), jnp.float32),
        pltpu.VMEM((NL1 * 16,), jnp.float32),
        pltpu.VMEM((CAP,), jnp.float32),
        pltpu.VMEM((CAP,), jnp.int32),
        pltpu.VMEM((64,), jnp.float32),
        pltpu.VMEM((64,), jnp.int32),
        pltpu.VMEM((64,), jnp.float32),
        pltpu.VMEM((64,), jnp.float32),
        pltpu.VMEM((16,), jnp.int32),
        pltpu.VMEM((64,), jnp.int32),
    ],
)
def _sc_sampler(lg_hbm, g_hbm, probs_hbm, tok_hbm,
                row_v, l1_v, cand_val, cand_idx, slot_val, slot_idx,
                g_v, probs_v, tok_v, vlist_v):
    cid = lax.axis_index("c")
    sid = lax.axis_index("s")
    wid = sid * 2 + cid
    lane = lax.broadcasted_iota(jnp.int32, (16,), 0)
    negv = jnp.full((16,), NEG, jnp.float32)

    def _row_body(rep, carry):
        row = wid + rep * 32
        pltpu.sync_copy(lg_hbm.at[row], row_v.at[pl.ds(0, V)])
        pltpu.sync_copy(g_hbm.at[pl.ds(row * 64, 64)], g_v)

        for i in range((VP - V) // 16):
            row_v[pl.ds(V + i * 16, 16)] = negv

        def _pa(g, c):
            base = g * 256
            vs = [row_v[pl.ds(base + k * 16, 16)] for k in range(16)]
            l1_v[pl.ds(g * 16, 16)] = _treemax(vs)
            return c
        lax.fori_loop(0, NL1, _pa, 0)

        l2 = []
        for h in range(NGROUP):
            vs = [l1_v[pl.ds((h * 16 + k) * 16, 16)] for k in range(16)]
            l2.append(_treemax(vs))

        def _round(r, state):
            t0, work = state[0], list(state[1:])
            t = plsc.cummax(_treemax(work))[15]
            tv = jnp.broadcast_to(t, (16,))
            work = [jnp.where(w == tv, NEG, w) for w in work]
            return (t, *work)
        t_c = lax.fori_loop(0, TOPK, _round, (jnp.float32(NEG), *l2))[0]
        tcv = jnp.broadcast_to(t_c, (16,))

        zero16 = jnp.zeros((16,), jnp.int32)

        def _pb(j, vcnt):
            x = l1_v[pl.ds(j * 16, 16)]
            m = x >= tcv
            csum = plsc.cumsum(m.astype(jnp.int32))
            pos = vcnt + csum - 1
            safe = jnp.logical_and(m, pos < 64)
            ids = jnp.broadcast_to(j * 16, (16,)) + lane
            plsc.store_scatter(vlist_v, [pos], ids, mask=safe)
            return vcnt + plsc.all_reduce_population_count(m)
        vcnt_vec = lax.fori_loop(0, NL1, _pb, zero16)
        vcnt = jnp.minimum(vcnt_vec[0], 64)

        for a in range(CAP // 16):
            cand_val[pl.ds(a * 16, 16)] = negv
            cand_idx[pl.ds(a * 16, 16)] = jnp.zeros((16,), jnp.int32)

        def _pc(e, ccnt):
            vv = plsc.load_gather(vlist_v, [jnp.broadcast_to(e, (16,))])
            gidx = (vv >> 4) * 256 + (vv & 15) + lane * 16
            x = plsc.load_gather(row_v, [gidx])
            m = x >= tcv
            csum = plsc.cumsum(m.astype(jnp.int32))
            pos = ccnt + csum - 1
            safe = jnp.logical_and(m, pos < CAP)
            plsc.store_scatter(cand_idx, [pos], gidx, mask=safe)
            plsc.store_scatter(cand_val, [pos], x, mask=safe)
            return ccnt + plsc.all_reduce_population_count(m)
        cnt_vec = lax.fori_loop(0, vcnt, _pc, zero16)

        cv = [cand_val[pl.ds(a * 16, 16)] for a in range(CAP // 16)]
        ci = [cand_idx[pl.ds(a * 16, 16)] for a in range(CAP // 16)]
        cnt = jnp.minimum(cnt_vec[0], CAP)

        def _rank(f, ranks):
            fv = jnp.broadcast_to(f, (16,))
            vf = plsc.load_gather(cand_val, [fv])
            jf = plsc.load_gather(cand_idx, [fv])
            out = []
            for a in range(CAP // 16):
                beat = (vf > cv[a]) | ((vf == cv[a]) & (jf < ci[a]))
                out.append(ranks[a] + beat.astype(jnp.int32))
            return tuple(out)
        ranks = lax.fori_loop(
            0, cnt, _rank,
            tuple(jnp.zeros((16,), jnp.int32) for _ in range(CAP // 16)))

        for a in range(4):
            slot_val[pl.ds(a * 16, 16)] = negv
            slot_idx[pl.ds(a * 16, 16)] = jnp.zeros((16,), jnp.int32)
        for a in range(CAP // 16):
            sel = ranks[a] < 64
            plsc.store_scatter(slot_val, [ranks[a]], cv[a], mask=sel)
            plsc.store_scatter(slot_idx, [ranks[a]], ci[a], mask=sel)

        v0v = jnp.broadcast_to(slot_val[pl.ds(0, 16)][0], (16,))
        sv = [slot_val[pl.ds(a * 16, 16)] for a in range(4)]
        siv = [slot_idx[pl.ds(a * 16, 16)] for a in range(4)]
        gvv = [g_v[pl.ds(a * 16, 16)] for a in range(4)]

        psum = jnp.zeros((16,), jnp.float32)
        pvecs = []
        for a in range(4):
            glob = lane + a * 16
            p = jnp.exp(sv[a] - v0v)
            p = jnp.where(glob < TOPK, p, jnp.float32(0.0))
            pvecs.append(p)
            psum = psum + p
        Sv = jnp.broadcast_to(plsc.cumsum(psum)[15], (16,))
        for a in range(4):
            probs_v[pl.ds(a * 16, 16)] = pvecs[a] / Sv

        mvec = negv
        scs = []
        for a in range(4):
            glob = lane + a * 16
            s = (sv[a] - v0v) + gvv[a]
            s = jnp.where(glob < TOPK, s, NEG)
            scs.append(s)
            mvec = jnp.maximum(mvec, s)
        msv = jnp.broadcast_to(plsc.cummax(mvec)[15], (16,))

        selv = jnp.full((16,), 9999, jnp.int32)
        for a in range(4):
            glob = lane + a * 16
            selv = jnp.minimum(selv, jnp.where(scs[a] == msv, glob, 9999))
        sel_i = jnp.broadcast_to(-plsc.cummax(-selv)[15], (16,))

        tokv = jnp.zeros((16,), jnp.int32)
        for a in range(4):
            glob = lane + a * 16
            tokv = tokv + jnp.where(glob == sel_i, siv[a], 0)
        tok_v[pl.ds(0, 16)] = jnp.broadcast_to(plsc.cumsum(tokv)[15], (16,))

        pltpu.sync_copy(probs_v, probs_hbm.at[pl.ds(row * 64, 64)])
        pltpu.sync_copy(tok_v, tok_hbm.at[pl.ds(row * 16, 16)])
        return carry

    lax.fori_loop(0, 2, _row_body, 0)


def kernel(logits):
    lgflat = logits[:, -1]
    g = jax.random.gumbel(jax.random.key(1234), (B, TOPK), jnp.float32)
    gp = jnp.zeros((B, 64), jnp.float32).at[:, :TOPK].set(g).reshape(-1)
    probs_out, tok_out = _sc_sampler(lgflat, gp)
    return (tok_out.reshape(B, 16)[:, 0],
            probs_out.reshape(B, 64)[:, :TOPK])


# speedup vs baseline: 1.9292x; 1.0404x over previous
import functools

import jax
import jax.numpy as jnp
from jax import lax
from jax.experimental import pallas as pl
from jax.experimental.pallas import tpu as pltpu
from jax.experimental.pallas import tpu_sc as plsc

B = 64
V = 100000
VP = 102400
NVREG = VP // 16
NL1 = 400
NGROUP = 25
TOPK = 50
CAP = 128
NEG = -3.4028235e38

_mesh = plsc.VectorSubcoreMesh(core_axis_name="c", subcore_axis_name="s")


def _treemax(vs):
    vs = list(vs)
    while len(vs) > 1:
        nxt = [jnp.maximum(vs[i], vs[i + 1]) for i in range(0, len(vs) - 1, 2)]
        if len(vs) % 2:
            nxt.append(vs[-1])
        vs = nxt
    return vs[0]


@functools.partial(
    pl.kernel,
    mesh=_mesh,
    compiler_params=pltpu.CompilerParams(
        needs_layout_passes=False, use_tc_tiling_on_sc=False),
    out_type=(
        jax.ShapeDtypeStruct((B // 2 * 64,), jnp.float32),
        jax.ShapeDtypeStruct((B // 2 * 16,), jnp.int32),
    ),
    scratch_types=[
        pltpu.VMEM((VP,), jnp.float32),
        pltpu.VMEM((NL1 * 16,), jnp.float32),
        pltpu.VMEM((CAP,), jnp.float32),
        pltpu.VMEM((CAP,), jnp.int32),
        pltpu.VMEM((64,), jnp.float32),
        pltpu.VMEM((64,), jnp.int32),
        pltpu.VMEM((64,), jnp.float32),
        pltpu.VMEM((64,), jnp.float32),
        pltpu.VMEM((16,), jnp.int32),
        pltpu.VMEM((64,), jnp.int32),
    ],
)
def _sc_sampler(lg_hbm, g_hbm, probs_hbm, tok_hbm,
                row_v, l1_v, cand_val, cand_idx, slot_val, slot_idx,
                g_v, probs_v, tok_v, vlist_v):
    cid = lax.axis_index("c")
    sid = lax.axis_index("s")
    wid = sid * 2 + cid
    lane = lax.broadcasted_iota(jnp.int32, (16,), 0)
    negv = jnp.full((16,), NEG, jnp.float32)

    if True:
        row = wid
        pltpu.sync_copy(lg_hbm.at[row], row_v.at[pl.ds(0, V)])
        pltpu.sync_copy(g_hbm.at[pl.ds(row * 64, 64)], g_v)

        for i in range((VP - V) // 16):
            row_v[pl.ds(V + i * 16, 16)] = negv

        def _pa(g, c):
            base = g * 256
            vs = [row_v[pl.ds(base + k * 16, 16)] for k in range(16)]
            l1_v[pl.ds(g * 16, 16)] = _treemax(vs)
            return c
        lax.fori_loop(0, NL1, _pa, 0)

        l2 = []
        for h in range(NGROUP):
            vs = [l1_v[pl.ds((h * 16 + k) * 16, 16)] for k in range(16)]
            l2.append(_treemax(vs))

        def _round(r, state):
            t0, work = state[0], list(state[1:])
            t = plsc.cummax(_treemax(work))[15]
            tv = jnp.broadcast_to(t, (16,))
            work = [jnp.where(w == tv, NEG, w) for w in work]
            return (t, *work)
        t_c = lax.fori_loop(0, TOPK, _round, (jnp.float32(NEG), *l2))[0]
        tcv = jnp.broadcast_to(t_c, (16,))

        zero16 = jnp.zeros((16,), jnp.int32)

        def _pb(j, vcnt):
            x = l1_v[pl.ds(j * 16, 16)]
            m = x >= tcv
            csum = plsc.cumsum(m.astype(jnp.int32))
            pos = vcnt + csum - 1
            safe = jnp.logical_and(m, pos < 64)
            ids = jnp.broadcast_to(j * 16, (16,)) + lane
            plsc.store_scatter(vlist_v, [pos], ids, mask=safe)
            return vcnt + plsc.all_reduce_population_count(m)
        vcnt_vec = lax.fori_loop(0, NL1, _pb, zero16)
        vcnt = jnp.minimum(vcnt_vec[0], 64)

        for a in range(CAP // 16):
            cand_val[pl.ds(a * 16, 16)] = negv
            cand_idx[pl.ds(a * 16, 16)] = jnp.zeros((16,), jnp.int32)

        def _pc(e, ccnt):
            vv = plsc.load_gather(vlist_v, [jnp.broadcast_to(e, (16,))])
            gidx = (vv >> 4) * 256 + (vv & 15) + lane * 16
            x = plsc.load_gather(row_v, [gidx])
            m = x >= tcv
            csum = plsc.cumsum(m.astype(jnp.int32))
            pos = ccnt + csum - 1
            safe = jnp.logical_and(m, pos < CAP)
            plsc.store_scatter(cand_idx, [pos], gidx, mask=safe)
            plsc.store_scatter(cand_val, [pos], x, mask=safe)
            return ccnt + plsc.all_reduce_population_count(m)
        cnt_vec = lax.fori_loop(0, vcnt, _pc, zero16)

        cv = [cand_val[pl.ds(a * 16, 16)] for a in range(CAP // 16)]
        ci = [cand_idx[pl.ds(a * 16, 16)] for a in range(CAP // 16)]
        cnt = jnp.minimum(cnt_vec[0], CAP)

        def _rank(f, ranks):
            fv = jnp.broadcast_to(f, (16,))
            vf = plsc.load_gather(cand_val, [fv])
            jf = plsc.load_gather(cand_idx, [fv])
            out = []
            for a in range(CAP // 16):
                beat = (vf > cv[a]) | ((vf == cv[a]) & (jf < ci[a]))
                out.append(ranks[a] + beat.astype(jnp.int32))
            return tuple(out)
        ranks = lax.fori_loop(
            0, cnt, _rank,
            tuple(jnp.zeros((16,), jnp.int32) for _ in range(CAP // 16)))

        for a in range(4):
            slot_val[pl.ds(a * 16, 16)] = negv
            slot_idx[pl.ds(a * 16, 16)] = jnp.zeros((16,), jnp.int32)
        for a in range(CAP // 16):
            sel = ranks[a] < 64
            plsc.store_scatter(slot_val, [ranks[a]], cv[a], mask=sel)
            plsc.store_scatter(slot_idx, [ranks[a]], ci[a], mask=sel)

        v0v = jnp.broadcast_to(slot_val[pl.ds(0, 16)][0], (16,))
        sv = [slot_val[pl.ds(a * 16, 16)] for a in range(4)]
        siv = [slot_idx[pl.ds(a * 16, 16)] for a in range(4)]
        gvv = [g_v[pl.ds(a * 16, 16)] for a in range(4)]

        psum = jnp.zeros((16,), jnp.float32)
        pvecs = []
        for a in range(4):
            glob = lane + a * 16
            p = jnp.exp(sv[a] - v0v)
            p = jnp.where(glob < TOPK, p, jnp.float32(0.0))
            pvecs.append(p)
            psum = psum + p
        Sv = jnp.broadcast_to(plsc.cumsum(psum)[15], (16,))
        for a in range(4):
            probs_v[pl.ds(a * 16, 16)] = pvecs[a] / Sv

        mvec = negv
        scs = []
        for a in range(4):
            glob = lane + a * 16
            s = (sv[a] - v0v) + gvv[a]
            s = jnp.where(glob < TOPK, s, NEG)
            scs.append(s)
            mvec = jnp.maximum(mvec, s)
        msv = jnp.broadcast_to(plsc.cummax(mvec)[15], (16,))

        selv = jnp.full((16,), 9999, jnp.int32)
        for a in range(4):
            glob = lane + a * 16
            selv = jnp.minimum(selv, jnp.where(scs[a] == msv, glob, 9999))
        sel_i = jnp.broadcast_to(-plsc.cummax(-selv)[15], (16,))

        tokv = jnp.zeros((16,), jnp.int32)
        for a in range(4):
            glob = lane + a * 16
            tokv = tokv + jnp.where(glob == sel_i, siv[a], 0)
        tok_v[pl.ds(0, 16)] = jnp.broadcast_to(plsc.cumsum(tokv)[15], (16,))

        pltpu.sync_copy(probs_v, probs_hbm.at[pl.ds(row * 64, 64)])
        pltpu.sync_copy(tok_v, tok_hbm.at[pl.ds(row * 16, 16)])


def kernel(logits):
    half = B // 2
    g = jax.random.gumbel(jax.random.key(1234), (B, TOPK), jnp.float32)
    gp = jnp.zeros((B, 64), jnp.float32).at[:, :TOPK].set(g)
    lg0 = logits[:half, -1]
    lg1 = logits[half:, -1]
    p0, t0 = _sc_sampler(lg0, gp[:half].reshape(-1))
    p1, t1 = _sc_sampler(lg1, gp[half:].reshape(-1))
    toks = jnp.concatenate([t0.reshape(half, 16)[:, 0],
                            t1.reshape(half, 16)[:, 0]])
    probs = jnp.concatenate([p0.reshape(half, 64)[:, :TOPK],
                             p1.reshape(half, 64)[:, :TOPK]], axis=0)
    return toks, probs


# speedup vs baseline: 2.4840x; 1.2876x over previous
import functools

import jax
import jax.numpy as jnp
from jax import lax
from jax.experimental import pallas as pl
from jax.experimental.pallas import tpu as pltpu
from jax.experimental.pallas import tpu_sc as plsc

B = 64
V = 100000
VP = 102400
NVREG = VP // 16
NL1 = 400
NGROUP = 25
TOPK = 50
CAP = 128
NEG = -3.4028235e38

_mesh = plsc.VectorSubcoreMesh(core_axis_name="c", subcore_axis_name="s")


def _treemax(vs):
    vs = list(vs)
    while len(vs) > 1:
        nxt = [jnp.maximum(vs[i], vs[i + 1]) for i in range(0, len(vs) - 1, 2)]
        if len(vs) % 2:
            nxt.append(vs[-1])
        vs = nxt
    return vs[0]


def _slice_body(i_ref, o_ref):
    o_ref[pl.ds(0, V)] = i_ref[7, :]
    o_ref[pl.ds(V, VP - V)] = jnp.full((VP - V,), NEG, jnp.float32)


def _last_step_padded(lg2d):
    return pl.pallas_call(
        _slice_body,
        grid=(B,),
        in_specs=[pl.BlockSpec((8, V), lambda r: (r, 0))],
        out_specs=pl.BlockSpec((VP,), lambda r: (r,)),
        out_shape=jax.ShapeDtypeStruct((B * VP,), jnp.float32),
    )(lg2d)


@functools.partial(
    pl.kernel,
    mesh=_mesh,
    compiler_params=pltpu.CompilerParams(
        needs_layout_passes=False, use_tc_tiling_on_sc=False),
    out_type=(
        jax.ShapeDtypeStruct((B * 64,), jnp.float32),
        jax.ShapeDtypeStruct((B * 16,), jnp.int32),
    ),
    scratch_types=[
        pltpu.VMEM((VP,), jnp.float32),
        pltpu.VMEM((NL1 * 16,), jnp.float32),
        pltpu.VMEM((CAP,), jnp.float32),
        pltpu.VMEM((CAP,), jnp.int32),
        pltpu.VMEM((64,), jnp.float32),
        pltpu.VMEM((64,), jnp.int32),
        pltpu.VMEM((64,), jnp.float32),
        pltpu.VMEM((64,), jnp.float32),
        pltpu.VMEM((16,), jnp.int32),
        pltpu.VMEM((64,), jnp.int32),
    ],
)
def _sc_sampler(lg_hbm, g_hbm, probs_hbm, tok_hbm,
                row_v, l1_v, cand_val, cand_idx, slot_val, slot_idx,
                g_v, probs_v, tok_v, vlist_v):
    cid = lax.axis_index("c")
    sid = lax.axis_index("s")
    wid = sid * 2 + cid
    lane = lax.broadcasted_iota(jnp.int32, (16,), 0)
    negv = jnp.full((16,), NEG, jnp.float32)

    def _row_body(rep, carry):
        row = wid + rep * 32
        pltpu.sync_copy(lg_hbm.at[pl.ds(row * VP, VP)], row_v)
        pltpu.sync_copy(g_hbm.at[pl.ds(row * 64, 64)], g_v)

        def _pa(g, c):
            base = g * 256
            vs = [row_v[pl.ds(base + k * 16, 16)] for k in range(16)]
            l1_v[pl.ds(g * 16, 16)] = _treemax(vs)
            return c
        lax.fori_loop(0, NL1, _pa, 0)

        l2 = []
        for h in range(NGROUP):
            vs = [l1_v[pl.ds((h * 16 + k) * 16, 16)] for k in range(16)]
            l2.append(_treemax(vs))

        def _round(r, state):
            t0, work = state[0], list(state[1:])
            t = plsc.cummax(_treemax(work))[15]
            tv = jnp.broadcast_to(t, (16,))
            work = [jnp.where(w == tv, NEG, w) for w in work]
            return (t, *work)
        t_c = lax.fori_loop(0, TOPK, _round, (jnp.float32(NEG), *l2))[0]
        tcv = jnp.broadcast_to(t_c, (16,))

        zero16 = jnp.zeros((16,), jnp.int32)

        def _pb(j, vcnt):
            x = l1_v[pl.ds(j * 16, 16)]
            m = x >= tcv
            csum = plsc.cumsum(m.astype(jnp.int32))
            pos = vcnt + csum - 1
            safe = jnp.logical_and(m, pos < 64)
            ids = jnp.broadcast_to(j * 16, (16,)) + lane
            plsc.store_scatter(vlist_v, [pos], ids, mask=safe)
            return vcnt + plsc.all_reduce_population_count(m)
        vcnt_vec = lax.fori_loop(0, NL1, _pb, zero16)
        vcnt = jnp.minimum(vcnt_vec[0], 64)

        for a in range(CAP // 16):
            cand_val[pl.ds(a * 16, 16)] = negv
            cand_idx[pl.ds(a * 16, 16)] = jnp.zeros((16,), jnp.int32)

        def _pc(e, ccnt):
            vv = plsc.load_gather(vlist_v, [jnp.broadcast_to(e, (16,))])
            gidx = (vv >> 4) * 256 + (vv & 15) + lane * 16
            x = plsc.load_gather(row_v, [gidx])
            m = x >= tcv
            csum = plsc.cumsum(m.astype(jnp.int32))
            pos = ccnt + csum - 1
            safe = jnp.logical_and(m, pos < CAP)
            plsc.store_scatter(cand_idx, [pos], gidx, mask=safe)
            plsc.store_scatter(cand_val, [pos], x, mask=safe)
            return ccnt + plsc.all_reduce_population_count(m)
        cnt_vec = lax.fori_loop(0, vcnt, _pc, zero16)

        cv = [cand_val[pl.ds(a * 16, 16)] for a in range(CAP // 16)]
        ci = [cand_idx[pl.ds(a * 16, 16)] for a in range(CAP // 16)]
        cnt = jnp.minimum(cnt_vec[0], CAP)

        def _rank(f, ranks):
            fv = jnp.broadcast_to(f, (16,))
            vf = plsc.load_gather(cand_val, [fv])
            jf = plsc.load_gather(cand_idx, [fv])
            out = []
            for a in range(CAP // 16):
                beat = (vf > cv[a]) | ((vf == cv[a]) & (jf < ci[a]))
                out.append(ranks[a] + beat.astype(jnp.int32))
            return tuple(out)
        ranks = lax.fori_loop(
            0, cnt, _rank,
            tuple(jnp.zeros((16,), jnp.int32) for _ in range(CAP // 16)))

        for a in range(4):
            slot_val[pl.ds(a * 16, 16)] = negv
            slot_idx[pl.ds(a * 16, 16)] = jnp.zeros((16,), jnp.int32)
        for a in range(CAP // 16):
            sel = ranks[a] < 64
            plsc.store_scatter(slot_val, [ranks[a]], cv[a], mask=sel)
            plsc.store_scatter(slot_idx, [ranks[a]], ci[a], mask=sel)

        v0v = jnp.broadcast_to(slot_val[pl.ds(0, 16)][0], (16,))
        sv = [slot_val[pl.ds(a * 16, 16)] for a in range(4)]
        siv = [slot_idx[pl.ds(a * 16, 16)] for a in range(4)]
        gvv = [g_v[pl.ds(a * 16, 16)] for a in range(4)]

        psum = jnp.zeros((16,), jnp.float32)
        pvecs = []
        for a in range(4):
            glob = lane + a * 16
            p = jnp.exp(sv[a] - v0v)
            p = jnp.where(glob < TOPK, p, jnp.float32(0.0))
            pvecs.append(p)
            psum = psum + p
        Sv = jnp.broadcast_to(plsc.cumsum(psum)[15], (16,))
        for a in range(4):
            probs_v[pl.ds(a * 16, 16)] = pvecs[a] / Sv

        mvec = negv
        scs = []
        for a in range(4):
            glob = lane + a * 16
            s = (sv[a] - v0v) + gvv[a]
            s = jnp.where(glob < TOPK, s, NEG)
            scs.append(s)
            mvec = jnp.maximum(mvec, s)
        msv = jnp.broadcast_to(plsc.cummax(mvec)[15], (16,))

        selv = jnp.full((16,), 9999, jnp.int32)
        for a in range(4):
            glob = lane + a * 16
            selv = jnp.minimum(selv, jnp.where(scs[a] == msv, glob, 9999))
        sel_i = jnp.broadcast_to(-plsc.cummax(-selv)[15], (16,))

        tokv = jnp.zeros((16,), jnp.int32)
        for a in range(4):
            glob = lane + a * 16
            tokv = tokv + jnp.where(glob == sel_i, siv[a], 0)
        tok_v[pl.ds(0, 16)] = jnp.broadcast_to(plsc.cumsum(tokv)[15], (16,))

        pltpu.sync_copy(probs_v, probs_hbm.at[pl.ds(row * 64, 64)])
        pltpu.sync_copy(tok_v, tok_hbm.at[pl.ds(row * 16, 16)])
        return carry

    lax.fori_loop(0, 2, _row_body, 0)


def kernel(logits):
    lg2d = logits.reshape(B * 8, V)
    lgflat = _last_step_padded(lg2d)
    g = jax.random.gumbel(jax.random.key(1234), (B, TOPK), jnp.float32)
    gp = jnp.zeros((B, 64), jnp.float32).at[:, :TOPK].set(g).reshape(-1)
    probs_out, tok_out = _sc_sampler(lgflat, gp)
    return (tok_out.reshape(B, 16)[:, 0],
            probs_out.reshape(B, 64)[:, :TOPK])


# speedup vs baseline: 2.7614x; 1.1117x over previous
import functools

import jax
import jax.numpy as jnp
from jax import lax
from jax.experimental import pallas as pl
from jax.experimental.pallas import tpu as pltpu
from jax.experimental.pallas import tpu_sc as plsc

B = 64
V = 100000
VP = 102400
NVREG = VP // 16
NL1 = 400
NGROUP = 25
TOPK = 50
CAP = 128
NEG = -3.4028235e38

_mesh = plsc.VectorSubcoreMesh(core_axis_name="c", subcore_axis_name="s")


def _treemax(vs):
    vs = list(vs)
    while len(vs) > 1:
        nxt = [jnp.maximum(vs[i], vs[i + 1]) for i in range(0, len(vs) - 1, 2)]
        if len(vs) % 2:
            nxt.append(vs[-1])
        vs = nxt
    return vs[0]


def _slice_body(i_ref, o_ref):
    o_ref[pl.ds(0, V)] = i_ref[7, :]
    o_ref[pl.ds(V, VP - V)] = jnp.full((VP - V,), NEG, jnp.float32)


def _last_step_padded(lg2d, off):
    return pl.pallas_call(
        _slice_body,
        grid=(B // 2,),
        in_specs=[pl.BlockSpec((8, V), lambda r: (r + off, 0))],
        out_specs=pl.BlockSpec((VP,), lambda r: (r,)),
        out_shape=jax.ShapeDtypeStruct((B // 2 * VP,), jnp.float32),
    )(lg2d)


@functools.partial(
    pl.kernel,
    mesh=_mesh,
    compiler_params=pltpu.CompilerParams(
        needs_layout_passes=False, use_tc_tiling_on_sc=False),
    out_type=(
        jax.ShapeDtypeStruct((B // 2 * 64,), jnp.float32),
        jax.ShapeDtypeStruct((B // 2 * 16,), jnp.int32),
    ),
    scratch_types=[
        pltpu.VMEM((VP,), jnp.float32),
        pltpu.VMEM((NL1 * 16,), jnp.float32),
        pltpu.VMEM((CAP,), jnp.float32),
        pltpu.VMEM((CAP,), jnp.int32),
        pltpu.VMEM((64,), jnp.float32),
        pltpu.VMEM((64,), jnp.int32),
        pltpu.VMEM((64,), jnp.float32),
        pltpu.VMEM((64,), jnp.float32),
        pltpu.VMEM((16,), jnp.int32),
        pltpu.VMEM((64,), jnp.int32),
    ],
)
def _sc_sampler(lg_hbm, g_hbm, probs_hbm, tok_hbm,
                row_v, l1_v, cand_val, cand_idx, slot_val, slot_idx,
                g_v, probs_v, tok_v, vlist_v):
    cid = lax.axis_index("c")
    sid = lax.axis_index("s")
    wid = sid * 2 + cid
    lane = lax.broadcasted_iota(jnp.int32, (16,), 0)
    negv = jnp.full((16,), NEG, jnp.float32)

    if True:
        row = wid
        pltpu.sync_copy(lg_hbm.at[pl.ds(row * VP, VP)], row_v)
        pltpu.sync_copy(g_hbm.at[pl.ds(row * 64, 64)], g_v)

        def _pa(g, c):
            base = g * 256
            vs = [row_v[pl.ds(base + k * 16, 16)] for k in range(16)]
            l1_v[pl.ds(g * 16, 16)] = _treemax(vs)
            return c
        lax.fori_loop(0, NL1, _pa, 0)

        l2 = []
        for h in range(NGROUP):
            vs = [l1_v[pl.ds((h * 16 + k) * 16, 16)] for k in range(16)]
            l2.append(_treemax(vs))

        def _round(r, state):
            t0, work = state[0], list(state[1:])
            t = plsc.cummax(_treemax(work))[15]
            tv = jnp.broadcast_to(t, (16,))
            work = [jnp.where(w == tv, NEG, w) for w in work]
            return (t, *work)
        t_c = lax.fori_loop(0, TOPK, _round, (jnp.float32(NEG), *l2))[0]
        tcv = jnp.broadcast_to(t_c, (16,))

        zero16 = jnp.zeros((16,), jnp.int32)

        def _pb(j, vcnt):
            x = l1_v[pl.ds(j * 16, 16)]
            m = x >= tcv
            csum = plsc.cumsum(m.astype(jnp.int32))
            pos = vcnt + csum - 1
            safe = jnp.logical_and(m, pos < 64)
            ids = jnp.broadcast_to(j * 16, (16,)) + lane
            plsc.store_scatter(vlist_v, [pos], ids, mask=safe)
            return vcnt + plsc.all_reduce_population_count(m)
        vcnt_vec = lax.fori_loop(0, NL1, _pb, zero16)
        vcnt = jnp.minimum(vcnt_vec[0], 64)

        for a in range(CAP // 16):
            cand_val[pl.ds(a * 16, 16)] = negv
            cand_idx[pl.ds(a * 16, 16)] = jnp.zeros((16,), jnp.int32)

        def _pc(e, ccnt):
            vv = plsc.load_gather(vlist_v, [jnp.broadcast_to(e, (16,))])
            gidx = (vv >> 4) * 256 + (vv & 15) + lane * 16
            x = plsc.load_gather(row_v, [gidx])
            m = x >= tcv
            csum = plsc.cumsum(m.astype(jnp.int32))
            pos = ccnt + csum - 1
            safe = jnp.logical_and(m, pos < CAP)
            plsc.store_scatter(cand_idx, [pos], gidx, mask=safe)
            plsc.store_scatter(cand_val, [pos], x, mask=safe)
            return ccnt + plsc.all_reduce_population_count(m)
        cnt_vec = lax.fori_loop(0, vcnt, _pc, zero16)

        cv = [cand_val[pl.ds(a * 16, 16)] for a in range(CAP // 16)]
        ci = [cand_idx[pl.ds(a * 16, 16)] for a in range(CAP // 16)]
        cnt = jnp.minimum(cnt_vec[0], CAP)

        def _rank(f, ranks):
            fv = jnp.broadcast_to(f, (16,))
            vf = plsc.load_gather(cand_val, [fv])
            jf = plsc.load_gather(cand_idx, [fv])
            out = []
            for a in range(CAP // 16):
                beat = (vf > cv[a]) | ((vf == cv[a]) & (jf < ci[a]))
                out.append(ranks[a] + beat.astype(jnp.int32))
            return tuple(out)
        ranks = lax.fori_loop(
            0, cnt, _rank,
            tuple(jnp.zeros((16,), jnp.int32) for _ in range(CAP // 16)))

        for a in range(4):
            slot_val[pl.ds(a * 16, 16)] = negv
            slot_idx[pl.ds(a * 16, 16)] = jnp.zeros((16,), jnp.int32)
        for a in range(CAP // 16):
            sel = ranks[a] < 64
            plsc.store_scatter(slot_val, [ranks[a]], cv[a], mask=sel)
            plsc.store_scatter(slot_idx, [ranks[a]], ci[a], mask=sel)

        v0v = jnp.broadcast_to(slot_val[pl.ds(0, 16)][0], (16,))
        sv = [slot_val[pl.ds(a * 16, 16)] for a in range(4)]
        siv = [slot_idx[pl.ds(a * 16, 16)] for a in range(4)]
        gvv = [g_v[pl.ds(a * 16, 16)] for a in range(4)]

        psum = jnp.zeros((16,), jnp.float32)
        pvecs = []
        for a in range(4):
            glob = lane + a * 16
            p = jnp.exp(sv[a] - v0v)
            p = jnp.where(glob < TOPK, p, jnp.float32(0.0))
            pvecs.append(p)
            psum = psum + p
        Sv = jnp.broadcast_to(plsc.cumsum(psum)[15], (16,))
        for a in range(4):
            probs_v[pl.ds(a * 16, 16)] = pvecs[a] / Sv

        mvec = negv
        scs = []
        for a in range(4):
            glob = lane + a * 16
            s = (sv[a] - v0v) + gvv[a]
            s = jnp.where(glob < TOPK, s, NEG)
            scs.append(s)
            mvec = jnp.maximum(mvec, s)
        msv = jnp.broadcast_to(plsc.cummax(mvec)[15], (16,))

        selv = jnp.full((16,), 9999, jnp.int32)
        for a in range(4):
            glob = lane + a * 16
            selv = jnp.minimum(selv, jnp.where(scs[a] == msv, glob, 9999))
        sel_i = jnp.broadcast_to(-plsc.cummax(-selv)[15], (16,))

        tokv = jnp.zeros((16,), jnp.int32)
        for a in range(4):
            glob = lane + a * 16
            tokv = tokv + jnp.where(glob == sel_i, siv[a], 0)
        tok_v[pl.ds(0, 16)] = jnp.broadcast_to(plsc.cumsum(tokv)[15], (16,))

        pltpu.sync_copy(probs_v, probs_hbm.at[pl.ds(row * 64, 64)])
        pltpu.sync_copy(tok_v, tok_hbm.at[pl.ds(row * 16, 16)])


def kernel(logits):
    half = B // 2
    lg2d = logits.reshape(B * 8, V)
    g = jax.random.gumbel(jax.random.key(1234), (B, TOPK), jnp.float32)
    gp = jnp.zeros((B, 64), jnp.float32).at[:, :TOPK].set(g)
    f0 = _last_step_padded(lg2d, 0)
    f1 = _last_step_padded(lg2d, half)
    p0, t0 = _sc_sampler(f0, gp[:half].reshape(-1))
    p1, t1 = _sc_sampler(f1, gp[half:].reshape(-1))
    toks = jnp.concatenate([t0.reshape(half, 16)[:, 0],
                            t1.reshape(half, 16)[:, 0]])
    probs = jnp.concatenate([p0.reshape(half, 64)[:, :TOPK],
                             p1.reshape(half, 64)[:, :TOPK]], axis=0)
    return toks, probs
